# baseline (speedup 1.0000x reference)
import jax
import jax.numpy as jnp
from jax import lax
from jax.experimental import pallas as pl
from jax.experimental.pallas import tpu as pltpu
from jax.experimental.pallas import tpu_sc as plsc

_N_FIELDS = 13
_FIELDS_PAD = 16
_VOCAB = 100000
_EMB = 32
_BATCH = 16384
_LANES = 16

_NC = 2
_NS = 16
_NW = _NC * _NS
_ROWS_PER_W = _BATCH // _NW
_CHUNK = 64
_N_CHUNKS = _ROWS_PER_W // _CHUNK
_IDX_ROWS = _CHUNK * _FIELDS_PAD // 128
_CROWS = _CHUNK * _N_FIELDS


def _body(idx_hbm, tab_hbm, out_hbm, idx_v, rows_v, outb_v, sem):
    wid = lax.axis_index("s") * _NC + lax.axis_index("c")

    def step(t, carry):
        pltpu.sync_copy(
            idx_hbm.at[pl.ds(wid * (_ROWS_PER_W * _FIELDS_PAD // 128)
                             + t * _IDX_ROWS, _IDX_ROWS)], idx_v)
        for b in range(_CHUNK):
            base = b * _FIELDS_PAD
            v0 = idx_v[base // 128, pl.ds(base % 128, _LANES)]
            for f in range(_N_FIELDS):
                pltpu.async_copy(
                    tab_hbm.at[pl.ds(v0[f], 1), :],
                    rows_v.at[pl.ds(b * _N_FIELDS + f, 1)], sem)
        pltpu.make_async_copy(tab_hbm.at[pl.ds(0, _CROWS)], rows_v, sem).wait()
        for b in range(_CHUNK):
            for h in range(_EMB // _LANES):
                sl = pl.ds(h * _LANES, _LANES)
                acc = rows_v[b * _N_FIELDS, sl]
                for f in range(1, _N_FIELDS):
                    acc = acc + rows_v[b * _N_FIELDS + f, sl]
                outb_v[b, sl] = acc
        pltpu.sync_copy(
            outb_v, out_hbm.at[pl.ds(wid * _ROWS_PER_W + t * _CHUNK, _CHUNK)])
        return carry

    lax.fori_loop(0, _N_CHUNKS, step, 0)


def _half(idxp, tab):
    run = pl.kernel(
        _body,
        mesh=plsc.VectorSubcoreMesh(core_axis_name="c", subcore_axis_name="s"),
        compiler_params=pltpu.CompilerParams(use_tc_tiling_on_sc=True),
        out_type=jax.ShapeDtypeStruct((_BATCH, _EMB), jnp.float32),
        scratch_types=[
            pltpu.VMEM((_IDX_ROWS, 128), jnp.int32),
            pltpu.VMEM((_CROWS, _EMB), jnp.float32),
            pltpu.VMEM((_CHUNK, _EMB), jnp.float32),
            pltpu.SemaphoreType.DMA,
        ],
    )
    return run(idxp, tab)


def _pack_idx(xh):
    offs = (jnp.arange(_N_FIELDS, dtype=jnp.int32) * _VOCAB)[None, :]
    flat = xh.astype(jnp.int32) + offs
    idxp = jnp.concatenate(
        [flat, jnp.zeros((_BATCH, _FIELDS_PAD - _N_FIELDS), jnp.int32)],
        axis=1)
    return idxp.reshape(_BATCH * _FIELDS_PAD // 128, 128)


@jax.jit
def kernel(x, tables):
    outa = _half(_pack_idx(x[:, :_N_FIELDS]),
                 tables[:_N_FIELDS].reshape(_N_FIELDS * _VOCAB, _EMB))
    outb = _half(_pack_idx(x[:, _N_FIELDS:]),
                 tables[_N_FIELDS:].reshape(_N_FIELDS * _VOCAB, _EMB))
    return outa + outb


# speedup vs baseline: 2.1518x; 2.1518x over previous
import jax
import jax.numpy as jnp
from jax import lax
from jax.experimental import pallas as pl
from jax.experimental.pallas import tpu as pltpu
from jax.experimental.pallas import tpu_sc as plsc

_N_FIELDS = 26
_FIELDS_PAD = 32
_VOCAB = 100000
_EMB = 32
_BATCH = 16384
_LANES = 16

_NC = 2
_NS = 16
_NW = _NC * _NS
_ROWS_PER_W = _BATCH // _NW
_CHUNK = 32
_N_CHUNKS = _ROWS_PER_W // _CHUNK
_IDX_ROWS = _CHUNK * _FIELDS_PAD // 128
_CROWS = _CHUNK * _N_FIELDS


def _body(idx_hbm, tab_hbm, out_hbm, idx_v, rows_v, outb_v, sem):
    wid = lax.axis_index("s") * _NC + lax.axis_index("c")

    def step(t, carry):
        pltpu.sync_copy(
            idx_hbm.at[pl.ds(wid * (_ROWS_PER_W * _FIELDS_PAD // 128)
                             + t * _IDX_ROWS, _IDX_ROWS)], idx_v)
        for b in range(_CHUNK):
            base = b * _FIELDS_PAD
            v0 = idx_v[base // 128, pl.ds(base % 128, _LANES)]
            v1 = idx_v[base // 128, pl.ds(base % 128 + _LANES, _LANES)]
            for f in range(_N_FIELDS):
                v = v0[f] if f < _LANES else v1[f - _LANES]
                pltpu.async_copy(
                    tab_hbm.at[pl.ds(v, 1), :],
                    rows_v.at[pl.ds(b * _N_FIELDS + f, 1)], sem)
        pltpu.make_async_copy(tab_hbm.at[pl.ds(0, _CROWS)], rows_v, sem).wait()
        for b in range(_CHUNK):
            for h in range(_EMB // _LANES):
                sl = pl.ds(h * _LANES, _LANES)
                acc = rows_v[b * _N_FIELDS, sl]
                for f in range(1, _N_FIELDS):
                    acc = acc + rows_v[b * _N_FIELDS + f, sl]
                outb_v[b, sl] = acc
        pltpu.sync_copy(
            outb_v, out_hbm.at[pl.ds(wid * _ROWS_PER_W + t * _CHUNK, _CHUNK)])
        return carry

    lax.fori_loop(0, _N_CHUNKS, step, 0)


@jax.jit
def kernel(x, tables):
    offs = (jnp.arange(_N_FIELDS, dtype=jnp.int32) * _VOCAB)[None, :]
    flat = x.astype(jnp.int32) + offs
    idxp = jnp.concatenate(
        [flat, jnp.zeros((_BATCH, _FIELDS_PAD - _N_FIELDS), jnp.int32)],
        axis=1).reshape(_BATCH * _FIELDS_PAD // 128, 128)
    tab2 = tables.reshape(_N_FIELDS * _VOCAB, _EMB)
    run = pl.kernel(
        _body,
        mesh=plsc.VectorSubcoreMesh(core_axis_name="c", subcore_axis_name="s"),
        compiler_params=pltpu.CompilerParams(use_tc_tiling_on_sc=True),
        out_type=jax.ShapeDtypeStruct((_BATCH, _EMB), jnp.float32),
        scratch_types=[
            pltpu.VMEM((_IDX_ROWS, 128), jnp.int32),
            pltpu.VMEM((_CROWS, _EMB), jnp.float32),
            pltpu.VMEM((_CHUNK, _EMB), jnp.float32),
            pltpu.SemaphoreType.DMA,
        ],
    )
    return run(idxp, tab2)
